# baseline (speedup 1.0000x reference)
import jax
import jax.numpy as jnp
from jax.experimental import pallas as pl
from jax.experimental.pallas import tpu as pltpu

_B = 16


def _edge_kernel(xa_ref, w1_ref, b1_ref, w2_ref, b2_ref, out_ref):
    w1 = w1_ref[...]
    bn = w1.shape[0] // 2
    w1a = w1[:bn, :]
    w1b = w1[bn:, :]
    b1r = b1_ref[...]
    w2r = w2_ref[...].astype(jnp.bfloat16)
    b2v = b2_ref[0, 0]
    H = w1a.shape[1]
    zero = jnp.bfloat16(0.0)

    def group(g, carry):
        for bi in range(4):
            b = g * 4 + bi
            x = xa_ref[b]
            a = jnp.dot(x, w1a, preferred_element_type=jnp.float32) + b1r
            b_t = jax.lax.dot_general(
                w1b, x, dimension_numbers=(((0,), (1,)), ((), ())),
                preferred_element_type=jnp.float32,
            )
            a16 = a.astype(jnp.bfloat16)
            bt16 = b_t.astype(jnp.bfloat16)
            acc0 = zero
            acc1 = zero
            for h in range(0, H, 2):
                t0 = jnp.maximum(a16[:, h:h + 1] + bt16[h:h + 1, :], zero)
                acc0 = acc0 + w2r[0:1, h:h + 1] * t0
                t1 = jnp.maximum(a16[:, h + 1:h + 2] + bt16[h + 1:h + 2, :],
                                 zero)
                acc1 = acc1 + w2r[0:1, h + 1:h + 2] * t1
            logits = (acc0 + acc1).astype(jnp.float32) + b2v
            out_ref[b] = jax.nn.sigmoid(logits)
        return carry

    jax.lax.fori_loop(0, _B // 4, group, 0, unroll=False)


def kernel(xa, W1, b1, W2, b2):
    B, C, BN = xa.shape
    H = W1.shape[1]
    b1r = b1.reshape(1, H)
    w2r = W2.reshape(1, H)
    b2s = b2.reshape(1, 1)
    return pl.pallas_call(
        _edge_kernel,
        grid=(1,),
        in_specs=[
            pl.BlockSpec((B, C, BN), lambda i: (0, 0, 0)),
            pl.BlockSpec((2 * BN, H), lambda i: (0, 0)),
            pl.BlockSpec((1, H), lambda i: (0, 0)),
            pl.BlockSpec((1, H), lambda i: (0, 0)),
            pl.BlockSpec((1, 1), lambda i: (0, 0)),
        ],
        out_specs=pl.BlockSpec((B, C, C), lambda i: (0, 0, 0)),
        out_shape=jax.ShapeDtypeStruct((B, C, C), jnp.float32),
    )(xa, W1, b1r, w2r, b2s)


# speedup vs baseline: 1.0687x; 1.0687x over previous
import jax
import jax.numpy as jnp
from jax.experimental import pallas as pl

_B = 16


def _edge_kernel(xa_ref, w1_ref, b1_ref, w2_ref, b2_ref, out_ref):
    w1 = w1_ref[...]
    bn = w1.shape[0] // 2
    w1a = w1[:bn, :]
    w1b = w1[bn:, :]
    b1r = b1_ref[...]
    w2r = w2_ref[...].astype(jnp.bfloat16)
    b2v = b2_ref[0, 0]
    H = w1a.shape[1]
    zero = jnp.bfloat16(0.0)
    for b in range(_B):
        x = xa_ref[b]
        a = jnp.dot(x, w1a, preferred_element_type=jnp.float32) + b1r
        b_t = jax.lax.dot_general(
            w1b, x, dimension_numbers=(((0,), (1,)), ((), ())),
            preferred_element_type=jnp.float32,
        )
        a16 = a.astype(jnp.bfloat16)
        bt16 = b_t.astype(jnp.bfloat16)
        acc0 = zero
        acc1 = zero
        for h in range(0, H, 2):
            t0 = jnp.maximum(a16[:, h:h + 1] + bt16[h:h + 1, :], zero)
            acc0 = acc0 + w2r[0:1, h:h + 1] * t0
            t1 = jnp.maximum(a16[:, h + 1:h + 2] + bt16[h + 1:h + 2, :], zero)
            acc1 = acc1 + w2r[0:1, h + 1:h + 2] * t1
        logits = (acc0 + acc1).astype(jnp.float32) + b2v
        out_ref[b] = jax.nn.sigmoid(logits)


def kernel(xa, W1, b1, W2, b2):
    B, C, BN = xa.shape
    H = W1.shape[1]
    b1r = b1.reshape(1, H)
    w2r = W2.reshape(1, H)
    b2s = b2.reshape(1, 1)
    return pl.pallas_call(
        _edge_kernel,
        grid=(1,),
        in_specs=[
            pl.BlockSpec((B, C, BN), lambda i: (0, 0, 0)),
            pl.BlockSpec((2 * BN, H), lambda i: (0, 0)),
            pl.BlockSpec((1, H), lambda i: (0, 0)),
            pl.BlockSpec((1, H), lambda i: (0, 0)),
            pl.BlockSpec((1, 1), lambda i: (0, 0)),
        ],
        out_specs=pl.BlockSpec((B, C, C), lambda i: (0, 0, 0)),
        out_shape=jax.ShapeDtypeStruct((B, C, C), jnp.float32),
    )(xa, W1, b1r, w2r, b2s)
